# baseline (speedup 1.0000x reference)
import jax
import jax.numpy as jnp
from jax.experimental import pallas as pl
from jax.experimental.pallas import tpu as pltpu

K = 16
TN = 128


def _softmax_k(lg):
    m = jnp.max(lg, axis=1, keepdims=True)
    e = jnp.exp(lg - m)
    return e * pl.reciprocal(jnp.sum(e, axis=1, keepdims=True), approx=True)


def _fused_kernel(x_ref, cg_ref,
                  w_if_ref, sc_if_ref, sh_if_ref,
                  w_lse_ref, sc_lse_ref, sh_lse_ref,
                  w_att1_ref, w_fc1_ref, b1_ref,
                  w_p1_ref, sc_p1_ref, sh_p1_ref,
                  w_att2_ref, w_fc2_ref, b2_ref,
                  w_p2_ref, sc_p2_ref, sh_p2_ref,
                  w_mo_ref, sc_mo_ref, sh_mo_ref,
                  o_ref):
    tn = x_ref.shape[0]

    y = jnp.dot(x_ref[...], w_if_ref[...], preferred_element_type=jnp.float32)
    y = jnp.maximum(y * sc_if_ref[...] + sh_if_ref[...], 0.0)
    residual = y[:, :256]
    fc1 = y[:, 256:384]

    lse = jnp.dot(cg_ref[...], w_lse_ref[...],
                  preferred_element_type=jnp.float32)
    lse = jnp.maximum(lse * sc_lse_ref[...] + sh_lse_ref[...], 0.0)
    lse1f = lse[:, :128]
    lse2f = lse[:, 128:]

    f1 = jnp.dot(fc1, w_fc1_ref[...],
                 preferred_element_type=jnp.float32) + b1_ref[...]
    lg1 = jnp.dot(lse1f, w_att1_ref[...],
                  preferred_element_type=jnp.float32)
    lg1 = lg1.reshape(tn, K, 256) + f1[:, None, :]
    s1 = _softmax_k(lg1)
    agg_a1 = jnp.sum(s1[:, :, :128] * lse1f.reshape(tn, K, 128), axis=1)
    agg_b1 = fc1 * jnp.sum(s1[:, :, 128:], axis=1)
    p1 = jnp.dot(jnp.concatenate([agg_a1, agg_b1], axis=1), w_p1_ref[...],
                 preferred_element_type=jnp.float32)
    att1 = jnp.maximum(p1 * sc_p1_ref[...] + sh_p1_ref[...], 0.0)

    f2 = jnp.dot(att1, w_fc2_ref[...],
                 preferred_element_type=jnp.float32) + b2_ref[...]
    lg2 = jnp.dot(lse2f, w_att2_ref[...],
                  preferred_element_type=jnp.float32)
    lg2 = lg2.reshape(tn, K, 256) + f2[:, None, :]
    s2 = _softmax_k(lg2)
    agg_a2 = jnp.sum(s2[:, :, :128] * lse2f.reshape(tn, K, 128), axis=1)
    agg_b2 = att1 * jnp.sum(s2[:, :, 128:], axis=1)
    p2 = jnp.dot(jnp.concatenate([agg_a2, agg_b2], axis=1), w_p2_ref[...],
                 preferred_element_type=jnp.float32)
    att2 = jnp.maximum(p2 * sc_p2_ref[...] + sh_p2_ref[...], 0.0)

    z = jnp.dot(att2, w_mo_ref[...], preferred_element_type=jnp.float32)
    z = jnp.maximum(z * sc_mo_ref[...] + sh_mo_ref[...], 0.0) + residual
    o_ref[...] = jnp.where(z >= 0.0, z, 0.01 * z)


def _const_spec(shape):
    return pl.BlockSpec(shape, lambda *_: (0,) * len(shape))


def _knn(xyz, k):
    d2 = jnp.sum((xyz[:, :, None, :] - xyz[:, None, :, :]) ** 2, axis=-1)
    neg, idx = jax.lax.top_k(-d2, k)
    return idx.astype(jnp.int32), -neg


def _fold_lse_w(w):
    return jnp.concatenate([
        w[0:3] + w[6:9],
        w[3:6] - w[6:9],
        w[9:10],
        jnp.zeros((1, w.shape[1]), w.dtype),
    ], axis=0)


def kernel(feat, xyz,
           in_fused_w, in_fused_sc, in_fused_sh,
           mlp_out_w, mlp_out_sc, mlp_out_sh,
           lse1_w, lse1_sc, lse1_sh,
           lse2_w, lse2_sc, lse2_sh,
           att1_w_aa, att1_w_ab, att1_w_ba, att1_w_bb, att1_b_a, att1_b_b,
           att2_w_aa, att2_w_ab, att2_w_ba, att2_w_bb, att2_b_a, att2_b_b,
           pool1_w_a, pool1_w_b, pool1_sc, pool1_sh,
           pool2_w_a, pool2_w_b, pool2_sc, pool2_sh):
    B, d_in, N, _ = feat.shape
    BN = B * N

    x = jnp.transpose(feat[..., 0], (0, 2, 1)).reshape(BN, d_in)

    idx, dist2 = _knn(xyz, K)
    center = jnp.broadcast_to(xyz[:, :, None, :], (B, N, K, 3))
    neigh = jax.vmap(lambda pts, ind: pts[ind])(xyz, idx)
    cg = jnp.concatenate(
        [center, neigh, dist2[..., None],
         jnp.zeros((B, N, K, 1), jnp.float32)], axis=-1)
    cg = cg.reshape(BN * K, 8)

    w_lse = jnp.concatenate([_fold_lse_w(lse1_w), _fold_lse_w(lse2_w)], axis=1)
    sc_lse = jnp.concatenate([lse1_sc, lse2_sc], axis=1)
    sh_lse = jnp.concatenate([lse1_sh, lse2_sh], axis=1)
    w_att1 = jnp.concatenate([att1_w_aa, att1_w_ab], axis=1)
    w_fc1 = jnp.concatenate([att1_w_ba, att1_w_bb], axis=1)
    b1 = jnp.concatenate([att1_b_a, att1_b_b], axis=1)
    w_att2 = jnp.concatenate([att2_w_aa, att2_w_ab], axis=1)
    w_fc2 = jnp.concatenate([att2_w_ba, att2_w_bb], axis=1)
    b2 = jnp.concatenate([att2_b_a, att2_b_b], axis=1)
    w_p1 = jnp.concatenate([pool1_w_a, pool1_w_b], axis=0)
    w_p2 = jnp.concatenate([pool2_w_a, pool2_w_b], axis=0)

    out = pl.pallas_call(
        _fused_kernel,
        out_shape=jax.ShapeDtypeStruct((BN, 256), jnp.float32),
        grid=(BN // TN,),
        in_specs=[
            pl.BlockSpec((TN, 128), lambda i: (i, 0)),
            pl.BlockSpec((TN * K, 8), lambda i: (i, 0)),
            _const_spec((128, 384)), _const_spec((1, 384)), _const_spec((1, 384)),
            _const_spec((8, 256)), _const_spec((1, 256)), _const_spec((1, 256)),
            _const_spec((128, 256)), _const_spec((128, 256)), _const_spec((1, 256)),
            _const_spec((256, 128)), _const_spec((1, 128)), _const_spec((1, 128)),
            _const_spec((128, 256)), _const_spec((128, 256)), _const_spec((1, 256)),
            _const_spec((256, 128)), _const_spec((1, 128)), _const_spec((1, 128)),
            _const_spec((128, 256)), _const_spec((1, 256)), _const_spec((1, 256)),
        ],
        out_specs=pl.BlockSpec((TN, 256), lambda i: (i, 0)),
        compiler_params=pltpu.CompilerParams(
            dimension_semantics=("parallel",),
            vmem_limit_bytes=100 * 1024 * 1024,
        ),
    )(x, cg,
      in_fused_w, in_fused_sc, in_fused_sh,
      w_lse, sc_lse, sh_lse,
      w_att1, w_fc1, b1, w_p1, pool1_sc, pool1_sh,
      w_att2, w_fc2, b2, w_p2, pool2_sc, pool2_sh,
      mlp_out_w, mlp_out_sc, mlp_out_sh)

    out = out.reshape(B, N, 256)
    return jnp.transpose(out, (0, 2, 1))[..., None]


# speedup vs baseline: 3.4921x; 3.4921x over previous
import jax
import jax.numpy as jnp
from jax.experimental import pallas as pl
from jax.experimental.pallas import tpu as pltpu

K = 16
TN = 128


def _softmax_k(lg):
    m = jnp.max(lg, axis=1, keepdims=True)
    e = jnp.exp(lg - m)
    return e * pl.reciprocal(jnp.sum(e, axis=1, keepdims=True), approx=True)


def _fused_kernel(x_ref, cg_ref,
                  w_if_ref, sc_if_ref, sh_if_ref,
                  w_lse_ref, sc_lse_ref, sh_lse_ref,
                  w_att1_ref, w_fc1_ref, b1_ref,
                  w_p1_ref, sc_p1_ref, sh_p1_ref,
                  w_att2_ref, w_fc2_ref, b2_ref,
                  w_p2_ref, sc_p2_ref, sh_p2_ref,
                  w_mo_ref, sc_mo_ref, sh_mo_ref,
                  o_ref):
    tn = x_ref.shape[0]

    y = jnp.dot(x_ref[...], w_if_ref[...], preferred_element_type=jnp.float32)
    y = jnp.maximum(y * sc_if_ref[...] + sh_if_ref[...], 0.0)
    residual = y[:, :256]
    fc1 = y[:, 256:384]

    lse = jnp.dot(cg_ref[...], w_lse_ref[...],
                  preferred_element_type=jnp.float32)
    lse = jnp.maximum(lse * sc_lse_ref[...] + sh_lse_ref[...], 0.0)
    lse1f = lse[:, :128]
    lse2f = lse[:, 128:]

    f1 = jnp.dot(fc1, w_fc1_ref[...],
                 preferred_element_type=jnp.float32) + b1_ref[...]
    lg1 = jnp.dot(lse1f, w_att1_ref[...],
                  preferred_element_type=jnp.float32)
    lg1 = lg1.reshape(tn, K, 256) + f1[:, None, :]
    s1 = _softmax_k(lg1)
    agg_a1 = jnp.sum(s1[:, :, :128] * lse1f.reshape(tn, K, 128), axis=1)
    agg_b1 = fc1 * jnp.sum(s1[:, :, 128:], axis=1)
    p1 = jnp.dot(jnp.concatenate([agg_a1, agg_b1], axis=1), w_p1_ref[...],
                 preferred_element_type=jnp.float32)
    att1 = jnp.maximum(p1 * sc_p1_ref[...] + sh_p1_ref[...], 0.0)

    f2 = jnp.dot(att1, w_fc2_ref[...],
                 preferred_element_type=jnp.float32) + b2_ref[...]
    lg2 = jnp.dot(lse2f, w_att2_ref[...],
                  preferred_element_type=jnp.float32)
    lg2 = lg2.reshape(tn, K, 256) + f2[:, None, :]
    s2 = _softmax_k(lg2)
    agg_a2 = jnp.sum(s2[:, :, :128] * lse2f.reshape(tn, K, 128), axis=1)
    agg_b2 = att1 * jnp.sum(s2[:, :, 128:], axis=1)
    p2 = jnp.dot(jnp.concatenate([agg_a2, agg_b2], axis=1), w_p2_ref[...],
                 preferred_element_type=jnp.float32)
    att2 = jnp.maximum(p2 * sc_p2_ref[...] + sh_p2_ref[...], 0.0)

    z = jnp.dot(att2, w_mo_ref[...], preferred_element_type=jnp.float32)
    z = jnp.maximum(z * sc_mo_ref[...] + sh_mo_ref[...], 0.0) + residual
    o_ref[...] = jnp.where(z >= 0.0, z, 0.01 * z)


def _const_spec(shape):
    return pl.BlockSpec(shape, lambda *_: (0,) * len(shape))


QT = 512


def _knn_kernel(xq_ref, pts_ref, o_ref):
    xq = xq_ref[0]
    px = pts_ref[0, 0:1, :]
    py = pts_ref[0, 1:2, :]
    pz = pts_ref[0, 2:3, :]
    dx = xq[:, 0:1] - px
    dy = xq[:, 1:2] - py
    dz = xq[:, 2:3] - pz
    d2 = dx * dx + dy * dy + dz * dz
    lane = jax.lax.broadcasted_iota(jnp.int32, d2.shape, 1)
    key = ((pltpu.bitcast(d2, jnp.int32) & ~jnp.int32(0xFFF)) | lane) \
        + jnp.int32(0x20000000)
    work = pltpu.bitcast(key, jnp.float32)
    bound = jnp.full((xq.shape[0], 1), -1.0, jnp.float32)
    cols = []
    for _ in range(K):
        cand = jnp.where(work > bound, work, jnp.inf)
        m = jnp.min(cand, axis=1, keepdims=True)
        cols.append(m)
        bound = m
    packed = jnp.concatenate(cols, axis=1)
    o_ref[0] = pltpu.bitcast(packed, jnp.int32) & jnp.int32(0xFFF)


def _knn(xyz):
    B, N, _ = xyz.shape
    qt = min(QT, N)
    xq = jnp.concatenate(
        [xyz, jnp.zeros((B, N, 5), jnp.float32)], axis=-1)
    pts = jnp.transpose(xq, (0, 2, 1))
    idx = pl.pallas_call(
        _knn_kernel,
        out_shape=jax.ShapeDtypeStruct((B, N, K), jnp.int32),
        grid=(B, N // qt),
        in_specs=[
            pl.BlockSpec((1, qt, 8), lambda b, i: (b, i, 0)),
            pl.BlockSpec((1, 8, N), lambda b, i: (b, 0, 0)),
        ],
        out_specs=pl.BlockSpec((1, qt, K), lambda b, i: (b, i, 0)),
        compiler_params=pltpu.CompilerParams(
            dimension_semantics=("parallel", "parallel"),
            vmem_limit_bytes=100 * 1024 * 1024,
        ),
    )(xq, pts)
    return idx


def _fold_lse_w(w):
    return jnp.concatenate([
        w[0:3] + w[6:9],
        w[3:6] - w[6:9],
        w[9:10],
        jnp.zeros((1, w.shape[1]), w.dtype),
    ], axis=0)


def kernel(feat, xyz,
           in_fused_w, in_fused_sc, in_fused_sh,
           mlp_out_w, mlp_out_sc, mlp_out_sh,
           lse1_w, lse1_sc, lse1_sh,
           lse2_w, lse2_sc, lse2_sh,
           att1_w_aa, att1_w_ab, att1_w_ba, att1_w_bb, att1_b_a, att1_b_b,
           att2_w_aa, att2_w_ab, att2_w_ba, att2_w_bb, att2_b_a, att2_b_b,
           pool1_w_a, pool1_w_b, pool1_sc, pool1_sh,
           pool2_w_a, pool2_w_b, pool2_sc, pool2_sh):
    B, d_in, N, _ = feat.shape
    BN = B * N

    x = jnp.transpose(feat[..., 0], (0, 2, 1)).reshape(BN, d_in)

    idx = _knn(xyz)
    center = jnp.broadcast_to(xyz[:, :, None, :], (B, N, K, 3))
    neigh = jax.vmap(lambda pts, ind: pts[ind])(xyz, idx)
    dist2 = jnp.sum((center - neigh) ** 2, axis=-1)
    cg = jnp.concatenate(
        [center, neigh, dist2[..., None],
         jnp.zeros((B, N, K, 1), jnp.float32)], axis=-1)
    cg = cg.reshape(BN * K, 8)

    w_lse = jnp.concatenate([_fold_lse_w(lse1_w), _fold_lse_w(lse2_w)], axis=1)
    sc_lse = jnp.concatenate([lse1_sc, lse2_sc], axis=1)
    sh_lse = jnp.concatenate([lse1_sh, lse2_sh], axis=1)
    w_att1 = jnp.concatenate([att1_w_aa, att1_w_ab], axis=1)
    w_fc1 = jnp.concatenate([att1_w_ba, att1_w_bb], axis=1)
    b1 = jnp.concatenate([att1_b_a, att1_b_b], axis=1)
    w_att2 = jnp.concatenate([att2_w_aa, att2_w_ab], axis=1)
    w_fc2 = jnp.concatenate([att2_w_ba, att2_w_bb], axis=1)
    b2 = jnp.concatenate([att2_b_a, att2_b_b], axis=1)
    w_p1 = jnp.concatenate([pool1_w_a, pool1_w_b], axis=0)
    w_p2 = jnp.concatenate([pool2_w_a, pool2_w_b], axis=0)

    out = pl.pallas_call(
        _fused_kernel,
        out_shape=jax.ShapeDtypeStruct((BN, 256), jnp.float32),
        grid=(BN // TN,),
        in_specs=[
            pl.BlockSpec((TN, 128), lambda i: (i, 0)),
            pl.BlockSpec((TN * K, 8), lambda i: (i, 0)),
            _const_spec((128, 384)), _const_spec((1, 384)), _const_spec((1, 384)),
            _const_spec((8, 256)), _const_spec((1, 256)), _const_spec((1, 256)),
            _const_spec((128, 256)), _const_spec((128, 256)), _const_spec((1, 256)),
            _const_spec((256, 128)), _const_spec((1, 128)), _const_spec((1, 128)),
            _const_spec((128, 256)), _const_spec((128, 256)), _const_spec((1, 256)),
            _const_spec((256, 128)), _const_spec((1, 128)), _const_spec((1, 128)),
            _const_spec((128, 256)), _const_spec((1, 256)), _const_spec((1, 256)),
        ],
        out_specs=pl.BlockSpec((TN, 256), lambda i: (i, 0)),
        compiler_params=pltpu.CompilerParams(
            dimension_semantics=("parallel",),
            vmem_limit_bytes=100 * 1024 * 1024,
        ),
    )(x, cg,
      in_fused_w, in_fused_sc, in_fused_sh,
      w_lse, sc_lse, sh_lse,
      w_att1, w_fc1, b1, w_p1, pool1_sc, pool1_sh,
      w_att2, w_fc2, b2, w_p2, pool2_sc, pool2_sh,
      mlp_out_w, mlp_out_sc, mlp_out_sh)

    out = out.reshape(B, N, 256)
    return jnp.transpose(out, (0, 2, 1))[..., None]


# speedup vs baseline: 10.6046x; 3.0367x over previous
import jax
import jax.numpy as jnp
from jax.experimental import pallas as pl
from jax.experimental.pallas import tpu as pltpu

K = 16
TN = 128


def _softmax_k(lg):
    m = jnp.max(lg, axis=1, keepdims=True)
    e = jnp.exp(lg - m)
    return e * pl.reciprocal(jnp.sum(e, axis=1, keepdims=True), approx=True)


def _fused_kernel(x_ref, xq_ref, idx_ref, ptsb_ref,
                  w_if_ref, sc_if_ref, sh_if_ref,
                  w_c_ref, w_n_ref, w_d_ref, sc_lse_ref, sh_lse_ref,
                  w_att1_ref, w_fc1_ref, b1_ref,
                  w_p1_ref, sc_p1_ref, sh_p1_ref,
                  w_att2_ref, w_fc2_ref, b2_ref,
                  w_p2_ref, sc_p2_ref, sh_p2_ref,
                  w_mo_ref, sc_mo_ref, sh_mo_ref,
                  o_ref, gath_ref):
    tn = x_ref.shape[0]

    def body(i, _):
        j = idx_ref[i >> 4, i & 15]
        gath_ref[i, :] = ptsb_ref[0, j, :]
        return 0
    jax.lax.fori_loop(0, tn * K, body, 0, unroll=8)

    y = jnp.dot(x_ref[...], w_if_ref[...], preferred_element_type=jnp.float32)
    y = jnp.maximum(y * sc_if_ref[...] + sh_if_ref[...], 0.0)
    residual = y[:, :256]
    fc1 = y[:, 256:384]

    g = gath_ref[...]
    xq = xq_ref[...]
    diff = g.reshape(tn, K, 8) - xq[:, None, :]
    d2 = jnp.sum(diff * diff, axis=2, keepdims=True)
    cproj = jnp.dot(xq, w_c_ref[...], preferred_element_type=jnp.float32)
    nproj = jnp.dot(g, w_n_ref[...], preferred_element_type=jnp.float32)
    lse = nproj.reshape(tn, K, 256) + cproj[:, None, :] + d2 * w_d_ref[0][None, None, :]
    lse = jnp.maximum(lse * sc_lse_ref[0][None, None, :]
                      + sh_lse_ref[0][None, None, :], 0.0)
    lse = lse.reshape(tn * K, 256)
    lse1f = lse[:, :128]
    lse2f = lse[:, 128:]

    f1 = jnp.dot(fc1, w_fc1_ref[...],
                 preferred_element_type=jnp.float32) + b1_ref[...]
    lg1 = jnp.dot(lse1f, w_att1_ref[...],
                  preferred_element_type=jnp.float32)
    lg1 = lg1.reshape(tn, K, 256) + f1[:, None, :]
    s1 = _softmax_k(lg1)
    agg_a1 = jnp.sum(s1[:, :, :128] * lse1f.reshape(tn, K, 128), axis=1)
    agg_b1 = fc1 * jnp.sum(s1[:, :, 128:], axis=1)
    p1 = jnp.dot(jnp.concatenate([agg_a1, agg_b1], axis=1), w_p1_ref[...],
                 preferred_element_type=jnp.float32)
    att1 = jnp.maximum(p1 * sc_p1_ref[...] + sh_p1_ref[...], 0.0)

    f2 = jnp.dot(att1, w_fc2_ref[...],
                 preferred_element_type=jnp.float32) + b2_ref[...]
    lg2 = jnp.dot(lse2f, w_att2_ref[...],
                  preferred_element_type=jnp.float32)
    lg2 = lg2.reshape(tn, K, 256) + f2[:, None, :]
    s2 = _softmax_k(lg2)
    agg_a2 = jnp.sum(s2[:, :, :128] * lse2f.reshape(tn, K, 128), axis=1)
    agg_b2 = att1 * jnp.sum(s2[:, :, 128:], axis=1)
    p2 = jnp.dot(jnp.concatenate([agg_a2, agg_b2], axis=1), w_p2_ref[...],
                 preferred_element_type=jnp.float32)
    att2 = jnp.maximum(p2 * sc_p2_ref[...] + sh_p2_ref[...], 0.0)

    z = jnp.dot(att2, w_mo_ref[...], preferred_element_type=jnp.float32)
    z = jnp.maximum(z * sc_mo_ref[...] + sh_mo_ref[...], 0.0) + residual
    o_ref[...] = jnp.where(z >= 0.0, z, 0.01 * z)


def _const_spec(shape):
    return pl.BlockSpec(shape, lambda *_: (0,) * len(shape))


QT = 512


def _knn_kernel(xq_ref, pts_ref, o_ref):
    xq = xq_ref[0]
    px = pts_ref[0, 0:1, :]
    py = pts_ref[0, 1:2, :]
    pz = pts_ref[0, 2:3, :]
    dx = xq[:, 0:1] - px
    dy = xq[:, 1:2] - py
    dz = xq[:, 2:3] - pz
    d2 = dx * dx + dy * dy + dz * dz
    lane = jax.lax.broadcasted_iota(jnp.int32, d2.shape, 1)
    key = ((pltpu.bitcast(d2, jnp.int32) & ~jnp.int32(0xFFF)) | lane) \
        + jnp.int32(0x20000000)
    work = pltpu.bitcast(key, jnp.float32)
    bound = jnp.full((xq.shape[0], 1), -1.0, jnp.float32)
    cols = []
    for _ in range(K):
        cand = jnp.where(work > bound, work, jnp.inf)
        m = jnp.min(cand, axis=1, keepdims=True)
        cols.append(m)
        bound = m
    packed = jnp.concatenate(cols, axis=1)
    o_ref[0] = pltpu.bitcast(packed, jnp.int32) & jnp.int32(0xFFF)


def _knn(xyz, xq):
    B, N, _ = xyz.shape
    qt = min(QT, N)
    pts = jnp.transpose(xq, (0, 2, 1))
    idx = pl.pallas_call(
        _knn_kernel,
        out_shape=jax.ShapeDtypeStruct((B, N, K), jnp.int32),
        grid=(B, N // qt),
        in_specs=[
            pl.BlockSpec((1, qt, 8), lambda b, i: (b, i, 0)),
            pl.BlockSpec((1, 8, N), lambda b, i: (b, 0, 0)),
        ],
        out_specs=pl.BlockSpec((1, qt, K), lambda b, i: (b, i, 0)),
        compiler_params=pltpu.CompilerParams(
            dimension_semantics=("parallel", "parallel"),
            vmem_limit_bytes=100 * 1024 * 1024,
        ),
    )(xq, pts)
    return idx


def kernel(feat, xyz,
           in_fused_w, in_fused_sc, in_fused_sh,
           mlp_out_w, mlp_out_sc, mlp_out_sh,
           lse1_w, lse1_sc, lse1_sh,
           lse2_w, lse2_sc, lse2_sh,
           att1_w_aa, att1_w_ab, att1_w_ba, att1_w_bb, att1_b_a, att1_b_b,
           att2_w_aa, att2_w_ab, att2_w_ba, att2_w_bb, att2_b_a, att2_b_b,
           pool1_w_a, pool1_w_b, pool1_sc, pool1_sh,
           pool2_w_a, pool2_w_b, pool2_sc, pool2_sh):
    B, d_in, N, _ = feat.shape
    BN = B * N

    x = jnp.transpose(feat[..., 0], (0, 2, 1)).reshape(BN, d_in)

    xq_b = jnp.concatenate(
        [xyz, jnp.zeros((B, N, 5), jnp.float32)], axis=-1)
    idx = _knn(xyz, xq_b).reshape(BN, K)
    xq = xq_b.reshape(BN, 8)

    def _parts(w):
        zc = jnp.zeros((5, w.shape[1]), w.dtype)
        w_c = jnp.concatenate([w[0:3] + w[6:9], zc], axis=0)
        w_n = jnp.concatenate([w[3:6] - w[6:9], zc], axis=0)
        return w_c, w_n, w[9:10]
    c1, n1, d1 = _parts(lse1_w)
    c2, n2, d2w = _parts(lse2_w)
    w_c = jnp.concatenate([c1, c2], axis=1)
    w_n = jnp.concatenate([n1, n2], axis=1)
    w_d = jnp.concatenate([d1, d2w], axis=1)
    sc_lse = jnp.concatenate([lse1_sc, lse2_sc], axis=1)
    sh_lse = jnp.concatenate([lse1_sh, lse2_sh], axis=1)
    w_att1 = jnp.concatenate([att1_w_aa, att1_w_ab], axis=1)
    w_fc1 = jnp.concatenate([att1_w_ba, att1_w_bb], axis=1)
    b1 = jnp.concatenate([att1_b_a, att1_b_b], axis=1)
    w_att2 = jnp.concatenate([att2_w_aa, att2_w_ab], axis=1)
    w_fc2 = jnp.concatenate([att2_w_ba, att2_w_bb], axis=1)
    b2 = jnp.concatenate([att2_b_a, att2_b_b], axis=1)
    w_p1 = jnp.concatenate([pool1_w_a, pool1_w_b], axis=0)
    w_p2 = jnp.concatenate([pool2_w_a, pool2_w_b], axis=0)

    nb = N // TN
    out = pl.pallas_call(
        _fused_kernel,
        out_shape=jax.ShapeDtypeStruct((BN, 256), jnp.float32),
        grid=(BN // TN,),
        in_specs=[
            pl.BlockSpec((TN, 128), lambda i: (i, 0)),
            pl.BlockSpec((TN, 8), lambda i: (i, 0)),
            pl.BlockSpec((TN, K), lambda i: (i, 0),
                         memory_space=pltpu.SMEM),
            pl.BlockSpec((1, N, 8), lambda i: (i // nb, 0, 0)),
            _const_spec((128, 384)), _const_spec((1, 384)), _const_spec((1, 384)),
            _const_spec((8, 256)), _const_spec((8, 256)), _const_spec((1, 256)),
            _const_spec((1, 256)), _const_spec((1, 256)),
            _const_spec((128, 256)), _const_spec((128, 256)), _const_spec((1, 256)),
            _const_spec((256, 128)), _const_spec((1, 128)), _const_spec((1, 128)),
            _const_spec((128, 256)), _const_spec((128, 256)), _const_spec((1, 256)),
            _const_spec((256, 128)), _const_spec((1, 128)), _const_spec((1, 128)),
            _const_spec((128, 256)), _const_spec((1, 256)), _const_spec((1, 256)),
        ],
        out_specs=pl.BlockSpec((TN, 256), lambda i: (i, 0)),
        scratch_shapes=[pltpu.VMEM((TN * K, 8), jnp.float32)],
        compiler_params=pltpu.CompilerParams(
            dimension_semantics=("parallel",),
            vmem_limit_bytes=100 * 1024 * 1024,
        ),
    )(x, xq, idx, xq_b.reshape(B, N, 8),
      in_fused_w, in_fused_sc, in_fused_sh,
      w_c, w_n, w_d, sc_lse, sh_lse,
      w_att1, w_fc1, b1, w_p1, pool1_sc, pool1_sh,
      w_att2, w_fc2, b2, w_p2, pool2_sc, pool2_sh,
      mlp_out_w, mlp_out_sc, mlp_out_sh)

    out = out.reshape(B, N, 256)
    return jnp.transpose(out, (0, 2, 1))[..., None]
